# baseline (speedup 1.0000x reference)
import jax
import jax.numpy as jnp
from jax import lax
from jax.experimental import pallas as pl
from jax.experimental.pallas import tpu as pltpu
from jax.experimental.pallas import tpu_sc as plsc

_N = 10000
_NPAD = 10240
_E = 320000
_FIN = 128
_FO = 64
_F = _FIN + _FO
_NC, _NS, _L = 2, 16, 16
_NW = _NC * _NS
_EPW = _E // _NW
_FH = _F // 2
_EPT = _E // _NS
_K = 80
_RPT = _NPAD // _NS

_sc_mesh = plsc.VectorSubcoreMesh(
    core_axis_name="c", subcore_axis_name="s", num_cores=_NC, num_subcores=_NS)


def _deg_body(dst_hbm, w_hbm, out_hbm, dst_v, w_v, acc_v):
    c = lax.axis_index("c")
    s = lax.axis_index("s")
    wid = s * _NC + c
    base = pl.multiple_of(wid * _EPW, 8)
    pltpu.sync_copy(dst_hbm.at[pl.ds(base, _EPW)], dst_v)
    pltpu.sync_copy(w_hbm.at[pl.ds(base, _EPW)], w_v)
    zeros = jnp.zeros((_L,), jnp.float32)

    def zbody(i, carry):
        acc_v[pl.ds(i * _L, _L)] = zeros
        return carry

    lax.fori_loop(0, _NPAD // _L, zbody, 0)

    lane = lax.iota(jnp.int32, _L)
    prev_idx = jnp.maximum(lane - 1, 0)
    next_idx = jnp.minimum(lane + 1, _L - 1)

    def ebody(i, carry):
        idx = dst_v[pl.ds(i * _L, _L)]
        wv = w_v[pl.ds(i * _L, _L)]
        sk, sw = plsc.sort_key_val(idx, wv)
        cs = plsc.cumsum(sw)
        excl = cs - sw
        sk_prev = sk.at[prev_idx].get(mode="promise_in_bounds")
        sk_next = sk.at[next_idx].get(mode="promise_in_bounds")
        newrun = (sk != sk_prev) | (lane == 0)
        start_ex = plsc.cummax(jnp.where(newrun, excl, 0.0))
        runsum = cs - start_ex
        islast = (sk != sk_next) | (lane == _L - 1)
        plsc.addupdate_scatter(acc_v, [sk], runsum, mask=islast)
        return carry

    lax.fori_loop(0, _EPW // _L, ebody, 0)
    pltpu.sync_copy(acc_v, out_hbm.at[wid])


_deg_call = pl.kernel(
    _deg_body,
    compiler_params=pltpu.CompilerParams(needs_layout_passes=False),
    out_type=jax.ShapeDtypeStruct((_NW, _NPAD), jnp.float32),
    mesh=_sc_mesh,
    scratch_types=[
        pltpu.VMEM((_EPW,), jnp.int32),
        pltpu.VMEM((_EPW,), jnp.float32),
        pltpu.VMEM((_NPAD,), jnp.float32),
    ],
)


def _prep_body(part_ref, x_ref, h_ref, dis_ref, xh0_ref, xh1_ref):
    deg = jnp.sum(part_ref[...], axis=1, keepdims=True)
    pos = deg > 0.0
    degc = jnp.where(pos, deg, 1.0)
    disr = jnp.where(pos, lax.rsqrt(degc), 0.0)
    dis_ref[...] = disr
    xh0_ref[...] = x_ref[:, :_FH] * disr
    xh1_ref[:, :_FIN - _FH] = x_ref[:, _FH:] * disr
    xh1_ref[:, _FIN - _FH:] = h_ref[...] * disr


def _prep_call(part_t, xp, hp):
    R = 512
    grid = _NPAD // R
    return pl.pallas_call(
        _prep_body,
        grid=(grid,),
        in_specs=[
            pl.BlockSpec((R, _NW), lambda j: (j, 0)),
            pl.BlockSpec((R, _FIN), lambda j: (j, 0)),
            pl.BlockSpec((R, _FO), lambda j: (j, 0)),
        ],
        out_specs=[
            pl.BlockSpec((R, 1), lambda j: (j, 0)),
            pl.BlockSpec((R, _FH), lambda j: (j, 0)),
            pl.BlockSpec((R, _FH), lambda j: (j, 0)),
        ],
        out_shape=[
            jax.ShapeDtypeStruct((_NPAD, 1), jnp.float32),
            jax.ShapeDtypeStruct((_NPAD, _FH), jnp.float32),
            jax.ShapeDtypeStruct((_NPAD, _FH), jnp.float32),
        ],
    )(part_t, xp, hp)


def _edge_body(src_hbm, dst_hbm, w_hbm, xh0_hbm, xh1_hbm, out_hbm,
               src_v, dst_v, w_v, rows2_v, idx_v, idx2_v, idx80_v, acc_sh,
               gsem0, gsem1, ssem, psem0, psem1):
    c = lax.axis_index("c")
    s = lax.axis_index("s")
    nblk = _EPT // _K
    ebase = pl.multiple_of(s * _EPT, 8)

    def _aux(b, p, sem, issue):
        boff = pl.multiple_of(ebase + b * _K, 8)
        for hbm, buf in ((src_hbm, src_v), (dst_hbm, dst_v), (w_hbm, w_v)):
            d = pltpu.make_async_copy(hbm.at[pl.ds(boff, _K)], buf.at[p], sem)
            if issue:
                d.start()
            else:
                d.wait()

    zeros = jnp.zeros((_L,), jnp.float32)
    for r in range(_K):
        for t in range(_FH // _L):
            rows2_v[0, r, pl.ds(t * _L, _L)] = zeros

    def zacc(i, carry):
        off = pl.multiple_of(s * _RPT + i * _K, 8)
        pltpu.sync_copy(rows2_v.at[0], acc_sh.at[pl.ds(off, _K)])
        return carry

    lax.fori_loop(0, _RPT // _K, zacc, 0)
    plsc.subcore_barrier()

    lane = lax.iota(jnp.int32, _L)
    prev_idx = jnp.maximum(lane - 1, 0)
    rots = [jnp.where(lane >= k, lane - k, lane - k + _L)
            for k in range(1, _L)]

    def _gather(b, buf, sem, issue):
        sidx = src_v.at[buf]

        @pl.when(c == 0)
        def _():
            d = pltpu.make_async_copy(xh0_hbm.at[sidx], rows2_v.at[buf], sem)
            if issue:
                d.start()
            else:
                d.wait()

        @pl.when(c == 1)
        def _():
            d = pltpu.make_async_copy(xh1_hbm.at[sidx], rows2_v.at[buf], sem)
            if issue:
                d.start()
            else:
                d.wait()

    def _process(b, buf):
        for j in range(_K // _L):
            nv16 = w_v[buf, pl.ds(j * _L, _L)]
            for u in range(_L):
                r = j * _L + u
                nb = jnp.full((_L,), nv16[u])
                for t in range(_FH // _L):
                    rows2_v[buf, r, pl.ds(t * _L, _L)] = (
                        rows2_v[buf, r, pl.ds(t * _L, _L)] * nb)

        dvs = [dst_v[buf, pl.ds(j * _L, _L)] for j in range(_K // _L)]
        dup = jnp.zeros((_L,), jnp.int32)
        for jb in range(_K // _L):
            for k in range(_L):
                if k == 0:
                    rbk = dvs[jb]
                else:
                    rbk = dvs[jb].at[rots[k - 1]].get(
                        mode="promise_in_bounds")
                    dup = dup | jnp.where(rbk == dvs[jb], 1, 0)
                for ja in range(jb):
                    dup = dup | jnp.where(rbk == dvs[ja], 1, 0)
        hasdup80 = jnp.max(dup)

        @pl.when(hasdup80 == 0)
        def _():
            for j in range(_K // _L):
                idx80_v[pl.ds(j * _L, _L)] = dvs[j]
            pltpu.async_copy(rows2_v.at[buf], acc_sh.at[idx80_v], ssem,
                             add=True).wait()

        @pl.when(hasdup80 > 0)
        def _():
            descs = []
            for j in range(_K // _L):
                d16 = dvs[j]
                rows16 = rows2_v.at[buf, pl.ds(j * _L, _L)]
                sk, _ = plsc.sort_key_val(d16, lane)
                sk_prev = sk.at[prev_idx].get(mode="promise_in_bounds")
                hasdup = jnp.max(
                    jnp.where((sk == sk_prev) & (lane >= 1), 1, 0))

                @pl.when(hasdup == 0)
                def _():
                    idx_v[j, :] = d16

                @pl.when(hasdup > 0)
                def _():
                    idx_v[j, :] = jnp.full((_L,), _NPAD, jnp.int32)

                    def wcond(al):
                        return jnp.max(al) > 0

                    def wbody(al):
                        dup_e = jnp.zeros((_L,), jnp.int32)
                        for sh in range(1, _L):
                            rot = jnp.maximum(lane - sh, 0)
                            dr = d16.at[rot].get(mode="promise_in_bounds")
                            ar = al.at[rot].get(mode="promise_in_bounds")
                            ok = (dr == d16) & (ar > 0) & (lane >= sh)
                            dup_e = dup_e | jnp.where(ok, 1, 0)
                        firsts = (al > 0) & (dup_e == 0)
                        idx2_v[...] = jnp.where(firsts, d16, _NPAD)
                        pltpu.sync_copy(rows16, acc_sh.at[idx2_v], add=True)
                        return jnp.where(firsts, 0, al)

                    lax.while_loop(wcond, wbody,
                                   jnp.ones((_L,), jnp.int32))

                descs.append(
                    pltpu.async_copy(rows16, acc_sh.at[idx_v.at[j]], ssem,
                                     add=True))
            for d in descs:
                d.wait()

    _aux(0, 0, psem0, True)
    _aux(1, 1, psem1, True)
    _aux(0, 0, psem0, False)
    _gather(0, 0, gsem0, True)

    def _step(b, p, q, gsp, gsq, psp, psq):
        @pl.when(b + 1 < nblk)
        def _():
            _aux(b + 1, q, psq, False)
            _gather(b + 1, q, gsq, True)

        _gather(b, p, gsp, False)
        _process(b, p)

        @pl.when(b + 2 < nblk)
        def _():
            _aux(b + 2, p, psp, True)

    def blk2(bp, carry):
        b0 = bp * 2
        _step(b0, 0, 1, gsem0, gsem1, psem0, psem1)
        _step(b0 + 1, 1, 0, gsem1, gsem0, psem1, psem0)
        return carry

    lax.fori_loop(0, nblk // 2, blk2, 0)
    plsc.subcore_barrier()
    ooff = pl.multiple_of(s * _RPT, 8)
    pltpu.sync_copy(acc_sh.at[pl.ds(ooff, _RPT)],
                    out_hbm.at[c, pl.ds(ooff, _RPT)])


_edge_call = pl.kernel(
    _edge_body,
    compiler_params=pltpu.CompilerParams(
        needs_layout_passes=False, use_tc_tiling_on_sc=False),
    out_type=jax.ShapeDtypeStruct((_NC, _NPAD, _FH), jnp.float32),
    mesh=_sc_mesh,
    scratch_types=[
        pltpu.VMEM((2, _K), jnp.int32),
        pltpu.VMEM((2, _K), jnp.int32),
        pltpu.VMEM((2, _K), jnp.float32),
        pltpu.VMEM((2, _K, _FH), jnp.float32),
        pltpu.VMEM((_K // _L, _L), jnp.int32),
        pltpu.VMEM((_L,), jnp.int32),
        pltpu.VMEM((_K,), jnp.int32),
        pltpu.VMEM_SHARED((_NPAD + _L, _FH), jnp.float32),
        pltpu.SemaphoreType.DMA,
        pltpu.SemaphoreType.DMA,
        pltpu.SemaphoreType.DMA,
        pltpu.SemaphoreType.DMA,
        pltpu.SemaphoreType.DMA,
    ],
)


def _dense_body(seg0, seg1, dis_ref, x_ref, h_ref, c_ref, wx0, wx1, wh0, wh1,
                b_ref, wci, wcf, wco, wl, bl_ref, out_ref, h_out, c_out):
    sneg = -dis_ref[...] * jnp.concatenate([seg0[...], seg1[...]], axis=1)
    tx = sneg[:, :_FIN]
    th = sneg[:, _FIN:]
    g = (jnp.dot(x_ref[...], wx0[...], preferred_element_type=jnp.float32)
         + jnp.dot(tx, wx1[...], preferred_element_type=jnp.float32)
         + jnp.dot(h_ref[...], wh0[...], preferred_element_type=jnp.float32)
         + jnp.dot(th, wh1[...], preferred_element_type=jnp.float32)
         + b_ref[...])
    cc = c_ref[...]
    gi = g[:, 0 * _FO:1 * _FO]
    gf = g[:, 1 * _FO:2 * _FO]
    gc = g[:, 2 * _FO:3 * _FO]
    go = g[:, 3 * _FO:4 * _FO]
    ig = jax.nn.sigmoid(gi + wci[...] * cc)
    fg = jax.nn.sigmoid(gf + wcf[...] * cc)
    tg = jnp.tanh(gc)
    cn = fg * cc + ig * tg
    og = jax.nn.sigmoid(go + wco[...] * cn)
    hn = og * jnp.tanh(cn)
    c_out[...] = cn
    h_out[...] = hn
    out_ref[...] = (jnp.sum(jax.nn.relu(hn) * wl[...], axis=1, keepdims=True)
                    + bl_ref[...])


def _dense_call(seg0, seg1, disp, xp, hp, cp, Wx0, Wx1, Wh0, Wh1, b2,
                wci2, wcf2, wco2, wl2, bl2):
    R = 512
    grid = _NPAD // R
    full = lambda j: (0, 0)
    return pl.pallas_call(
        _dense_body,
        grid=(grid,),
        in_specs=[
            pl.BlockSpec((R, _FH), lambda j: (j, 0)),
            pl.BlockSpec((R, _FH), lambda j: (j, 0)),
            pl.BlockSpec((R, 1), lambda j: (j, 0)),
            pl.BlockSpec((R, _FIN), lambda j: (j, 0)),
            pl.BlockSpec((R, _FO), lambda j: (j, 0)),
            pl.BlockSpec((R, _FO), lambda j: (j, 0)),
            pl.BlockSpec((_FIN, 4 * _FO), full),
            pl.BlockSpec((_FIN, 4 * _FO), full),
            pl.BlockSpec((_FO, 4 * _FO), full),
            pl.BlockSpec((_FO, 4 * _FO), full),
            pl.BlockSpec((1, 4 * _FO), full),
            pl.BlockSpec((1, _FO), full),
            pl.BlockSpec((1, _FO), full),
            pl.BlockSpec((1, _FO), full),
            pl.BlockSpec((1, _FO), full),
            pl.BlockSpec((1, 1), full),
        ],
        out_specs=[
            pl.BlockSpec((R, 1), lambda j: (j, 0)),
            pl.BlockSpec((R, _FO), lambda j: (j, 0)),
            pl.BlockSpec((R, _FO), lambda j: (j, 0)),
        ],
        out_shape=[
            jax.ShapeDtypeStruct((_NPAD, 1), jnp.float32),
            jax.ShapeDtypeStruct((_NPAD, _FO), jnp.float32),
            jax.ShapeDtypeStruct((_NPAD, _FO), jnp.float32),
        ],
    )(seg0, seg1, disp, xp, hp, cp, Wx0, Wx1, Wh0, Wh1, b2, wci2, wcf2,
      wco2, wl2, bl2)


def kernel(x, edge_index, edge_weight, h, c, Wx0, Wx1, Wh0, Wh1, b,
           w_ci, w_cf, w_co, Wl, bl):
    src = edge_index[0]
    dst = edge_index[1]
    pad = _NPAD - _N
    xp = jnp.pad(x, ((0, pad), (0, 0)))
    hp = jnp.pad(h, ((0, pad), (0, 0)))
    cp = jnp.pad(c, ((0, pad), (0, 0)))
    degpart = _deg_call(dst, edge_weight)
    dis, xh0, xh1 = _prep_call(degpart.T, xp, hp)
    segpart = _edge_call(src, dst, edge_weight, xh0, xh1)
    out, hn, cn = _dense_call(
        segpart[0], segpart[1], dis, xp, hp, cp, Wx0, Wx1, Wh0, Wh1,
        b.reshape(1, -1), w_ci.reshape(1, -1), w_cf.reshape(1, -1),
        w_co.reshape(1, -1), Wl.reshape(1, -1), bl.reshape(1, 1))
    return out[:_N], hn[:_N], cn[:_N]


# speedup vs baseline: 1.1602x; 1.1602x over previous
import jax
import jax.numpy as jnp
from jax import lax
from jax.experimental import pallas as pl
from jax.experimental.pallas import tpu as pltpu
from jax.experimental.pallas import tpu_sc as plsc

_N = 10000
_NPAD = 10240
_E = 320000
_FIN = 128
_FO = 64
_F = _FIN + _FO
_NC, _NS, _L = 2, 16, 16
_NW = _NC * _NS
_EPW = _E // _NW
_FH = _F // 2
_EPT = _E // _NS
_K = 80
_RPT = _NPAD // _NS

_sc_mesh = plsc.VectorSubcoreMesh(
    core_axis_name="c", subcore_axis_name="s", num_cores=_NC, num_subcores=_NS)


def _deg_body(dst_hbm, w_hbm, out_hbm, dst_v, w_v, acc_v):
    c = lax.axis_index("c")
    s = lax.axis_index("s")
    wid = s * _NC + c
    base = pl.multiple_of(wid * _EPW, 8)
    pltpu.sync_copy(dst_hbm.at[pl.ds(base, _EPW)], dst_v)
    pltpu.sync_copy(w_hbm.at[pl.ds(base, _EPW)], w_v)
    zeros = jnp.zeros((_L,), jnp.float32)

    def zbody(i, carry):
        acc_v[pl.ds(i * _L, _L)] = zeros
        return carry

    lax.fori_loop(0, _NPAD // _L, zbody, 0)

    lane = lax.iota(jnp.int32, _L)
    prev_idx = jnp.maximum(lane - 1, 0)
    next_idx = jnp.minimum(lane + 1, _L - 1)

    def ebody(i, carry):
        idx = dst_v[pl.ds(i * _L, _L)]
        wv = w_v[pl.ds(i * _L, _L)]
        sk, sw = plsc.sort_key_val(idx, wv)
        cs = plsc.cumsum(sw)
        excl = cs - sw
        sk_prev = sk.at[prev_idx].get(mode="promise_in_bounds")
        sk_next = sk.at[next_idx].get(mode="promise_in_bounds")
        newrun = (sk != sk_prev) | (lane == 0)
        start_ex = plsc.cummax(jnp.where(newrun, excl, 0.0))
        runsum = cs - start_ex
        islast = (sk != sk_next) | (lane == _L - 1)
        plsc.addupdate_scatter(acc_v, [sk], runsum, mask=islast)
        return carry

    lax.fori_loop(0, _EPW // _L, ebody, 0)
    pltpu.sync_copy(acc_v, out_hbm.at[wid])


_deg_call = pl.kernel(
    _deg_body,
    compiler_params=pltpu.CompilerParams(needs_layout_passes=False),
    out_type=jax.ShapeDtypeStruct((_NW, _NPAD), jnp.float32),
    mesh=_sc_mesh,
    scratch_types=[
        pltpu.VMEM((_EPW,), jnp.int32),
        pltpu.VMEM((_EPW,), jnp.float32),
        pltpu.VMEM((_NPAD,), jnp.float32),
    ],
)


def _prep_body(part_ref, x_ref, h_ref, dis_ref, xh0_ref, xh1_ref):
    deg = jnp.sum(part_ref[...], axis=1, keepdims=True)
    pos = deg > 0.0
    degc = jnp.where(pos, deg, 1.0)
    disr = jnp.where(pos, lax.rsqrt(degc), 0.0)
    dis_ref[...] = disr
    xh0_ref[...] = x_ref[:, :_FH] * disr
    xh1_ref[:, :_FIN - _FH] = x_ref[:, _FH:] * disr
    xh1_ref[:, _FIN - _FH:] = h_ref[...] * disr


def _prep_call(part_t, xp, hp):
    R = 512
    grid = _NPAD // R
    return pl.pallas_call(
        _prep_body,
        grid=(grid,),
        in_specs=[
            pl.BlockSpec((R, _NW), lambda j: (j, 0)),
            pl.BlockSpec((R, _FIN), lambda j: (j, 0)),
            pl.BlockSpec((R, _FO), lambda j: (j, 0)),
        ],
        out_specs=[
            pl.BlockSpec((R, 1), lambda j: (j, 0)),
            pl.BlockSpec((R, _FH), lambda j: (j, 0)),
            pl.BlockSpec((R, _FH), lambda j: (j, 0)),
        ],
        out_shape=[
            jax.ShapeDtypeStruct((_NPAD, 1), jnp.float32),
            jax.ShapeDtypeStruct((_NPAD, _FH), jnp.float32),
            jax.ShapeDtypeStruct((_NPAD, _FH), jnp.float32),
        ],
    )(part_t, xp, hp)


def _edge_body(src_hbm, dst_hbm, w_hbm, xh0_hbm, xh1_hbm, out_hbm,
               src_v, dst_v, w_v, rows2_v, idx_v, idx2_v, acc_sh,
               gsem0, gsem1, ssem, psem0, psem1):
    c = lax.axis_index("c")
    s = lax.axis_index("s")
    nblk = _EPT // _K
    ebase = pl.multiple_of(s * _EPT, 8)

    def _aux(b, p, sem, issue):
        boff = pl.multiple_of(ebase + b * _K, 8)
        for hbm, buf in ((src_hbm, src_v), (dst_hbm, dst_v), (w_hbm, w_v)):
            d = pltpu.make_async_copy(hbm.at[pl.ds(boff, _K)], buf.at[p], sem)
            if issue:
                d.start()
            else:
                d.wait()

    zeros = jnp.zeros((_L,), jnp.float32)
    for r in range(_K):
        for t in range(_FH // _L):
            rows2_v[0, r, pl.ds(t * _L, _L)] = zeros

    def zacc(i, carry):
        off = pl.multiple_of(s * _RPT + i * _K, 8)
        pltpu.sync_copy(rows2_v.at[0], acc_sh.at[pl.ds(off, _K)])
        return carry

    lax.fori_loop(0, _RPT // _K, zacc, 0)
    plsc.subcore_barrier()

    lane = lax.iota(jnp.int32, _L)
    prev_idx = jnp.maximum(lane - 1, 0)

    def _gather(b, buf, sem, issue):
        sidx = src_v.at[buf]

        @pl.when(c == 0)
        def _():
            d = pltpu.make_async_copy(xh0_hbm.at[sidx], rows2_v.at[buf], sem)
            if issue:
                d.start()
            else:
                d.wait()

        @pl.when(c == 1)
        def _():
            d = pltpu.make_async_copy(xh1_hbm.at[sidx], rows2_v.at[buf], sem)
            if issue:
                d.start()
            else:
                d.wait()

    def _process(b, buf):
        descs = []
        for j in range(_K // _L):
            nv16 = w_v[buf, pl.ds(j * _L, _L)]
            for u in range(_L):
                r = j * _L + u
                nb = jnp.full((_L,), nv16[u])
                for t in range(_FH // _L):
                    rows2_v[buf, r, pl.ds(t * _L, _L)] = (
                        rows2_v[buf, r, pl.ds(t * _L, _L)] * nb)

            d16 = dst_v[buf, pl.ds(j * _L, _L)]
            rows16 = rows2_v.at[buf, pl.ds(j * _L, _L)]
            sk, _ = plsc.sort_key_val(d16, lane)
            sk_prev = sk.at[prev_idx].get(mode="promise_in_bounds")
            hasdup = jnp.max(
                jnp.where((sk == sk_prev) & (lane >= 1), 1, 0))

            @pl.when(hasdup == 0)
            def _():
                idx_v[j, :] = d16

            @pl.when(hasdup > 0)
            def _():
                idx_v[j, :] = jnp.full((_L,), _NPAD, jnp.int32)

                def wcond(al):
                    return jnp.max(al) > 0

                def wbody(al):
                    dup_e = jnp.zeros((_L,), jnp.int32)
                    for sh in range(1, _L):
                        rot = jnp.maximum(lane - sh, 0)
                        dr = d16.at[rot].get(mode="promise_in_bounds")
                        ar = al.at[rot].get(mode="promise_in_bounds")
                        ok = (dr == d16) & (ar > 0) & (lane >= sh)
                        dup_e = dup_e | jnp.where(ok, 1, 0)
                    firsts = (al > 0) & (dup_e == 0)
                    idx2_v[...] = jnp.where(firsts, d16, _NPAD)
                    pltpu.sync_copy(rows16, acc_sh.at[idx2_v], add=True)
                    return jnp.where(firsts, 0, al)

                lax.while_loop(wcond, wbody, jnp.ones((_L,), jnp.int32))

            descs.append(
                pltpu.async_copy(rows16, acc_sh.at[idx_v.at[j]], ssem,
                                 add=True))
        for d in descs:
            d.wait()

    _aux(0, 0, psem0, True)
    _aux(1, 1, psem1, True)
    _aux(0, 0, psem0, False)
    _gather(0, 0, gsem0, True)

    def _step(b, p, q, gsp, gsq, psp, psq):
        @pl.when(b + 1 < nblk)
        def _():
            _aux(b + 1, q, psq, False)
            _gather(b + 1, q, gsq, True)

        _gather(b, p, gsp, False)
        _process(b, p)

        @pl.when(b + 2 < nblk)
        def _():
            _aux(b + 2, p, psp, True)

    def blk2(bp, carry):
        b0 = bp * 2
        _step(b0, 0, 1, gsem0, gsem1, psem0, psem1)
        _step(b0 + 1, 1, 0, gsem1, gsem0, psem1, psem0)
        return carry

    lax.fori_loop(0, nblk // 2, blk2, 0)
    plsc.subcore_barrier()
    ooff = pl.multiple_of(s * _RPT, 8)
    pltpu.sync_copy(acc_sh.at[pl.ds(ooff, _RPT)],
                    out_hbm.at[c, pl.ds(ooff, _RPT)])


_edge_call = pl.kernel(
    _edge_body,
    compiler_params=pltpu.CompilerParams(
        needs_layout_passes=False, use_tc_tiling_on_sc=False),
    out_type=jax.ShapeDtypeStruct((_NC, _NPAD, _FH), jnp.float32),
    mesh=_sc_mesh,
    scratch_types=[
        pltpu.VMEM((2, _K), jnp.int32),
        pltpu.VMEM((2, _K), jnp.int32),
        pltpu.VMEM((2, _K), jnp.float32),
        pltpu.VMEM((2, _K, _FH), jnp.float32),
        pltpu.VMEM((_K // _L, _L), jnp.int32),
        pltpu.VMEM((_L,), jnp.int32),
        pltpu.VMEM_SHARED((_NPAD + _L, _FH), jnp.float32),
        pltpu.SemaphoreType.DMA,
        pltpu.SemaphoreType.DMA,
        pltpu.SemaphoreType.DMA,
        pltpu.SemaphoreType.DMA,
        pltpu.SemaphoreType.DMA,
    ],
)


def _dense_body(seg0, seg1, dis_ref, x_ref, h_ref, c_ref, wx0, wx1, wh0, wh1,
                b_ref, wci, wcf, wco, wl, bl_ref, out_ref, h_out, c_out):
    sneg = -dis_ref[...] * jnp.concatenate([seg0[...], seg1[...]], axis=1)
    tx = sneg[:, :_FIN]
    th = sneg[:, _FIN:]
    g = (jnp.dot(x_ref[...], wx0[...], preferred_element_type=jnp.float32)
         + jnp.dot(tx, wx1[...], preferred_element_type=jnp.float32)
         + jnp.dot(h_ref[...], wh0[...], preferred_element_type=jnp.float32)
         + jnp.dot(th, wh1[...], preferred_element_type=jnp.float32)
         + b_ref[...])
    cc = c_ref[...]
    gi = g[:, 0 * _FO:1 * _FO]
    gf = g[:, 1 * _FO:2 * _FO]
    gc = g[:, 2 * _FO:3 * _FO]
    go = g[:, 3 * _FO:4 * _FO]
    ig = jax.nn.sigmoid(gi + wci[...] * cc)
    fg = jax.nn.sigmoid(gf + wcf[...] * cc)
    tg = jnp.tanh(gc)
    cn = fg * cc + ig * tg
    og = jax.nn.sigmoid(go + wco[...] * cn)
    hn = og * jnp.tanh(cn)
    c_out[...] = cn
    h_out[...] = hn
    out_ref[...] = (jnp.sum(jax.nn.relu(hn) * wl[...], axis=1, keepdims=True)
                    + bl_ref[...])


def _dense_call(seg0, seg1, disp, xp, hp, cp, Wx0, Wx1, Wh0, Wh1, b2,
                wci2, wcf2, wco2, wl2, bl2):
    R = 512
    grid = _NPAD // R
    full = lambda j: (0, 0)
    return pl.pallas_call(
        _dense_body,
        grid=(grid,),
        in_specs=[
            pl.BlockSpec((R, _FH), lambda j: (j, 0)),
            pl.BlockSpec((R, _FH), lambda j: (j, 0)),
            pl.BlockSpec((R, 1), lambda j: (j, 0)),
            pl.BlockSpec((R, _FIN), lambda j: (j, 0)),
            pl.BlockSpec((R, _FO), lambda j: (j, 0)),
            pl.BlockSpec((R, _FO), lambda j: (j, 0)),
            pl.BlockSpec((_FIN, 4 * _FO), full),
            pl.BlockSpec((_FIN, 4 * _FO), full),
            pl.BlockSpec((_FO, 4 * _FO), full),
            pl.BlockSpec((_FO, 4 * _FO), full),
            pl.BlockSpec((1, 4 * _FO), full),
            pl.BlockSpec((1, _FO), full),
            pl.BlockSpec((1, _FO), full),
            pl.BlockSpec((1, _FO), full),
            pl.BlockSpec((1, _FO), full),
            pl.BlockSpec((1, 1), full),
        ],
        out_specs=[
            pl.BlockSpec((R, 1), lambda j: (j, 0)),
            pl.BlockSpec((R, _FO), lambda j: (j, 0)),
            pl.BlockSpec((R, _FO), lambda j: (j, 0)),
        ],
        out_shape=[
            jax.ShapeDtypeStruct((_NPAD, 1), jnp.float32),
            jax.ShapeDtypeStruct((_NPAD, _FO), jnp.float32),
            jax.ShapeDtypeStruct((_NPAD, _FO), jnp.float32),
        ],
    )(seg0, seg1, disp, xp, hp, cp, Wx0, Wx1, Wh0, Wh1, b2, wci2, wcf2,
      wco2, wl2, bl2)


def kernel(x, edge_index, edge_weight, h, c, Wx0, Wx1, Wh0, Wh1, b,
           w_ci, w_cf, w_co, Wl, bl):
    src = edge_index[0]
    dst = edge_index[1]
    pad = _NPAD - _N
    xp = jnp.pad(x, ((0, pad), (0, 0)))
    hp = jnp.pad(h, ((0, pad), (0, 0)))
    cp = jnp.pad(c, ((0, pad), (0, 0)))
    degpart = _deg_call(dst, edge_weight)
    dis, xh0, xh1 = _prep_call(degpart.T, xp, hp)
    segpart = _edge_call(src, dst, edge_weight, xh0, xh1)
    out, hn, cn = _dense_call(
        segpart[0], segpart[1], dis, xp, hp, cp, Wx0, Wx1, Wh0, Wh1,
        b.reshape(1, -1), w_ci.reshape(1, -1), w_cf.reshape(1, -1),
        w_co.reshape(1, -1), Wl.reshape(1, -1), bl.reshape(1, 1))
    return out[:_N], hn[:_N], cn[:_N]
